# baseline (speedup 1.0000x reference)
import jax
import jax.numpy as jnp
from jax import lax
from jax.experimental import pallas as pl
from jax.experimental.pallas import tpu as pltpu
from jax.experimental.pallas import tpu_sc as plsc

B = 16384
D = 64
NC, NS, L = 2, 16, 16
NW = NC * NS
BPW = B // NW
CH = 128
NCH = BPW // CH
GPC = CH // L
PAD = L + 1

_MU = 3.53


def _svd_body(uid_hbm, iid_hbm, ub_hbm, ib_hbm, uf2_hbm, if2_hbm, out_hbm,
              uidx_v, iidx_v, uhalf_v, ihalf_v, upar_v, ipar_v,
              pu_v, qi_v, bu_v, bi_v, res_v, scr_v, bsem, *fsems):
  wid = lax.axis_index("s") * NC + lax.axis_index("c")
  base = wid * BPW

  for c in range(NCH):
    pltpu.sync_copy(uid_hbm.at[pl.ds(base + c * CH, CH)], uidx_v.at[c])
    pltpu.sync_copy(iid_hbm.at[pl.ds(base + c * CH, CH)], iidx_v.at[c])

  bias_h = []
  for c in range(NCH):
    sl = pl.ds(c * CH, CH)
    bias_h.append(pltpu.async_copy(ub_hbm.at[uidx_v.at[c]], bu_v.at[sl], bsem))
    bias_h.append(pltpu.async_copy(ib_hbm.at[iidx_v.at[c]], bi_v.at[sl], bsem))

  def prep(i, carry):
    cc = i // (CH // L)
    off = (i % (CH // L)) * L
    uv = uidx_v[cc, pl.ds(off, L)]
    iv = iidx_v[cc, pl.ds(off, L)]
    uhalf_v[cc, pl.ds(off, L)] = lax.shift_right_logical(uv, 1)
    ihalf_v[cc, pl.ds(off, L)] = lax.shift_right_logical(iv, 1)
    upar_v[pl.ds(i * L, L)] = lax.bitwise_and(uv, 1) * D
    ipar_v[pl.ds(i * L, L)] = lax.bitwise_and(iv, 1) * D
    return carry
  lax.fori_loop(0, BPW // L, prep, 0)

  def fire(c):
    buf = c % 2
    pltpu.async_copy(uf2_hbm.at[uhalf_v.at[c]], pu_v.at[buf], fsems[c])
    pltpu.async_copy(if2_hbm.at[ihalf_v.at[c]], qi_v.at[buf], fsems[c])

  lane = lax.iota(jnp.int32, L)
  col_idx = lane * PAD

  def compute(c):
    buf = c % 2
    def group_body(g, carry):
      row0 = g * L
      up16 = upar_v[pl.ds(c * CH + row0, L)]
      ip16 = ipar_v[pl.ds(c * CH + row0, L)]
      for rr in range(L):
        r = row0 + rr
        po = up16[rr]
        qo = ip16[rr]
        acc = (pu_v[buf, r, pl.ds(po, L)] * qi_v[buf, r, pl.ds(qo, L)])
        for k in range(1, D // L):
          acc = acc + (pu_v[buf, r, pl.ds(po + k * L, L)]
                       * qi_v[buf, r, pl.ds(qo + k * L, L)])
        scr_v[pl.ds(rr * PAD, L)] = acc
      dots0 = plsc.load_gather(scr_v, [col_idx])
      dots1 = plsc.load_gather(scr_v, [col_idx + 1])
      dots2 = plsc.load_gather(scr_v, [col_idx + 2])
      dots3 = plsc.load_gather(scr_v, [col_idx + 3])
      for l in range(4, L, 4):
        dots0 = dots0 + plsc.load_gather(scr_v, [col_idx + l])
        dots1 = dots1 + plsc.load_gather(scr_v, [col_idx + l + 1])
        dots2 = dots2 + plsc.load_gather(scr_v, [col_idx + l + 2])
        dots3 = dots3 + plsc.load_gather(scr_v, [col_idx + l + 3])
      dots = (dots0 + dots1) + (dots2 + dots3)
      sl = pl.ds(c * CH + row0, L)
      rating = jnp.float32(_MU) + bu_v[sl] + bi_v[sl] + dots
      rating = jnp.minimum(jnp.maximum(rating, jnp.float32(1.0)),
                           jnp.float32(5.0))
      res_v[sl] = rating
      return carry
    lax.fori_loop(0, GPC, group_body, 0)

  fire(0)
  for c in range(NCH):
    if c + 1 < NCH:
      fire(c + 1)
    pltpu.make_async_copy(uf2_hbm.at[uhalf_v.at[c]], pu_v.at[c % 2],
                          fsems[c]).wait()
    pltpu.make_async_copy(if2_hbm.at[ihalf_v.at[c]], qi_v.at[c % 2],
                          fsems[c]).wait()
    if c == 0:
      for h in bias_h:
        h.wait()
    compute(c)

  pltpu.sync_copy(res_v, out_hbm.at[pl.ds(base, BPW)])


@jax.jit
def kernel(user_ids, item_ids, user_bias, item_bias, user_factors,
           item_factors):
  nu, ni = user_factors.shape[0], item_factors.shape[0]
  mesh = plsc.VectorSubcoreMesh(core_axis_name="c", subcore_axis_name="s")
  run = pl.kernel(
      _svd_body,
      out_type=jax.ShapeDtypeStruct((B,), jnp.float32),
      mesh=mesh,
      compiler_params=pltpu.CompilerParams(needs_layout_passes=False,
                                           use_tc_tiling_on_sc=True),
      scratch_types=[
          pltpu.VMEM((NCH, CH), jnp.int32),
          pltpu.VMEM((NCH, CH), jnp.int32),
          pltpu.VMEM((NCH, CH), jnp.int32),
          pltpu.VMEM((NCH, CH), jnp.int32),
          pltpu.VMEM((BPW,), jnp.int32),
          pltpu.VMEM((BPW,), jnp.int32),
          pltpu.VMEM((2, CH, 2 * D), jnp.float32),
          pltpu.VMEM((2, CH, 2 * D), jnp.float32),
          pltpu.VMEM((BPW,), jnp.float32),
          pltpu.VMEM((BPW,), jnp.float32),
          pltpu.VMEM((BPW,), jnp.float32),
          pltpu.VMEM((L * PAD,), jnp.float32),
          pltpu.SemaphoreType.DMA,
          pltpu.SemaphoreType.DMA,
          pltpu.SemaphoreType.DMA,
          pltpu.SemaphoreType.DMA,
          pltpu.SemaphoreType.DMA,
      ],
  )
  return run(user_ids.astype(jnp.int32), item_ids.astype(jnp.int32),
             user_bias.reshape(-1), item_bias.reshape(-1),
             user_factors.reshape(nu // 2, 2 * D),
             item_factors.reshape(ni // 2, 2 * D))


# speedup vs baseline: 1.3592x; 1.3592x over previous
import jax
import jax.numpy as jnp
from jax import lax
from jax.experimental import pallas as pl
from jax.experimental.pallas import tpu as pltpu
from jax.experimental.pallas import tpu_sc as plsc

B = 16384
D = 64
NC, NS, L = 2, 16, 16
NW = NC * NS
BPW = B // NW
CH = 128
NCH = BPW // CH
WV = 16
NWV = BPW // WV
PAD = L + 1

_MU = 3.53


def _svd_body(uid_hbm, iid_hbm, ub_hbm, ib_hbm, uf_hbm, if_hbm, out_hbm,
              uidx_v, iidx_v, ur8_v, ir8_v, urr_v, irr_v,
              pu_v, qi_v, bu_v, bi_v, res_v, scr_v,
              bsem, usem0, isem0, usem1, isem1):
  wid = lax.axis_index("s") * NC + lax.axis_index("c")
  base = wid * BPW

  for c in range(NCH):
    pltpu.sync_copy(uid_hbm.at[pl.ds(base + c * CH, CH)], uidx_v.at[c])
    pltpu.sync_copy(iid_hbm.at[pl.ds(base + c * CH, CH)], iidx_v.at[c])

  bias_h = []
  for c in range(NCH):
    sl = pl.ds(c * CH, CH)
    bias_h.append(pltpu.async_copy(ub_hbm.at[uidx_v.at[c]], bu_v.at[sl], bsem))
    bias_h.append(pltpu.async_copy(ib_hbm.at[iidx_v.at[c]], bi_v.at[sl], bsem))

  def prep(i, carry):
    cc = i // (CH // L)
    off = (i % (CH // L)) * L
    uv = uidx_v[cc, pl.ds(off, L)]
    iv = iidx_v[cc, pl.ds(off, L)]
    sl = pl.ds(i * L, L)
    ur8_v[sl] = lax.bitwise_and(uv, -8)
    ir8_v[sl] = lax.bitwise_and(iv, -8)
    urr_v[sl] = lax.bitwise_and(uv, 7)
    irr_v[sl] = lax.bitwise_and(iv, 7)
    return carry
  lax.fori_loop(0, BPW // L, prep, 0)

  def fire(w, slot, usem, isem):
    u16 = ur8_v[pl.ds(w * WV, L)]
    i16 = ir8_v[pl.ds(w * WV, L)]
    for j in range(L):
      ru = pl.multiple_of(u16[j], 8)
      ri = pl.multiple_of(i16[j], 8)
      pltpu.async_copy(uf_hbm.at[pl.ds(ru, 8), pl.ds(0, D)],
                       pu_v.at[slot, j], usem)
      pltpu.async_copy(if_hbm.at[pl.ds(ri, 8), pl.ds(0, D)],
                       qi_v.at[slot, j], isem)

  def drain(slot, usem, isem):
    for j in range(WV):
      pltpu.make_async_copy(uf_hbm.at[pl.ds(0, 8), pl.ds(0, D)],
                            pu_v.at[slot, j], usem).wait()
      pltpu.make_async_copy(if_hbm.at[pl.ds(0, 8), pl.ds(0, D)],
                            qi_v.at[slot, j], isem).wait()

  lane = lax.iota(jnp.int32, L)
  col_idx = lane * PAD

  def compute(w, slot):
    row0 = w * WV
    ur16 = urr_v[pl.ds(row0, L)]
    ir16 = irr_v[pl.ds(row0, L)]
    for j in range(L):
      ru = ur16[j]
      ri = ir16[j]
      acc = (pu_v[slot, j, ru, pl.ds(0, L)]
             * qi_v[slot, j, ri, pl.ds(0, L)])
      for k in range(1, D // L):
        acc = acc + (pu_v[slot, j, ru, pl.ds(k * L, L)]
                     * qi_v[slot, j, ri, pl.ds(k * L, L)])
      scr_v[pl.ds(j * PAD, L)] = acc
    dots0 = plsc.load_gather(scr_v, [col_idx])
    dots1 = plsc.load_gather(scr_v, [col_idx + 1])
    dots2 = plsc.load_gather(scr_v, [col_idx + 2])
    dots3 = plsc.load_gather(scr_v, [col_idx + 3])
    for l in range(4, L, 4):
      dots0 = dots0 + plsc.load_gather(scr_v, [col_idx + l])
      dots1 = dots1 + plsc.load_gather(scr_v, [col_idx + l + 1])
      dots2 = dots2 + plsc.load_gather(scr_v, [col_idx + l + 2])
      dots3 = dots3 + plsc.load_gather(scr_v, [col_idx + l + 3])
    dots = (dots0 + dots1) + (dots2 + dots3)
    sl = pl.ds(row0, L)
    rating = jnp.float32(_MU) + bu_v[sl] + bi_v[sl] + dots
    rating = jnp.minimum(jnp.maximum(rating, jnp.float32(1.0)),
                         jnp.float32(5.0))
    res_v[sl] = rating

  fire(0, 0, usem0, isem0)
  for h in bias_h:
    h.wait()

  def pair_body(p, carry):
    w0 = p * 2
    fire(w0 + 1, 1, usem1, isem1)
    drain(0, usem0, isem0)
    compute(w0, 0)

    @pl.when(p < NWV // 2 - 1)
    def _():
      fire(w0 + 2, 0, usem0, isem0)
    drain(1, usem1, isem1)
    compute(w0 + 1, 1)
    return carry

  lax.fori_loop(0, NWV // 2, pair_body, 0)

  pltpu.sync_copy(res_v, out_hbm.at[pl.ds(base, BPW)])


@jax.jit
def kernel(user_ids, item_ids, user_bias, item_bias, user_factors,
           item_factors):
  mesh = plsc.VectorSubcoreMesh(core_axis_name="c", subcore_axis_name="s")
  run = pl.kernel(
      _svd_body,
      out_type=jax.ShapeDtypeStruct((B,), jnp.float32),
      mesh=mesh,
      compiler_params=pltpu.CompilerParams(needs_layout_passes=False,
                                           use_tc_tiling_on_sc=True),
      scratch_types=[
          pltpu.VMEM((NCH, CH), jnp.int32),
          pltpu.VMEM((NCH, CH), jnp.int32),
          pltpu.VMEM((BPW,), jnp.int32),
          pltpu.VMEM((BPW,), jnp.int32),
          pltpu.VMEM((BPW,), jnp.int32),
          pltpu.VMEM((BPW,), jnp.int32),
          pltpu.VMEM((2, WV, 8, D), jnp.float32),
          pltpu.VMEM((2, WV, 8, D), jnp.float32),
          pltpu.VMEM((BPW,), jnp.float32),
          pltpu.VMEM((BPW,), jnp.float32),
          pltpu.VMEM((BPW,), jnp.float32),
          pltpu.VMEM((L * PAD,), jnp.float32),
          pltpu.SemaphoreType.DMA,
          pltpu.SemaphoreType.DMA,
          pltpu.SemaphoreType.DMA,
          pltpu.SemaphoreType.DMA,
          pltpu.SemaphoreType.DMA,
      ],
  )
  return run(user_ids.astype(jnp.int32), item_ids.astype(jnp.int32),
             user_bias.reshape(-1), item_bias.reshape(-1),
             user_factors, item_factors)


# speedup vs baseline: 2.1243x; 1.5629x over previous
import jax
import jax.numpy as jnp
from jax import lax
from jax.experimental import pallas as pl
from jax.experimental.pallas import tpu as pltpu
from jax.experimental.pallas import tpu_sc as plsc

B = 16384
D = 64
NC, NS, L = 2, 16, 16
NW = NC * NS
BPW = B // NW
CH = 128
NCH = BPW // CH
WV = 16
NWV = BPW // WV
PAD = L + 1
HALF = 500000

_MU = 3.53


def _svd_body(uid_hbm, iid_hbm, ub_hbm, ib_hbm, uf_hbm, if_hbm, out_hbm,
              uidx_v, iidx_v, ur8_v, ir8_v, urr_v, irr_v, ihal_v,
              pu_v, qi_v, bu_v, bi_v, res_v, scr_v,
              bsem, usem0, isem0, usem1, isem1):
  wid = lax.axis_index("s") * NC + lax.axis_index("c")
  base = wid * BPW

  for c in range(NCH):
    pltpu.sync_copy(uid_hbm.at[pl.ds(base + c * CH, CH)], uidx_v.at[c])
    pltpu.sync_copy(iid_hbm.at[pl.ds(base + c * CH, CH)], iidx_v.at[c])

  bias_h = []
  for c in range(NCH):
    sl = pl.ds(c * CH, CH)
    bias_h.append(pltpu.async_copy(ub_hbm.at[uidx_v.at[c]], bu_v.at[sl], bsem))
    bias_h.append(pltpu.async_copy(ib_hbm.at[iidx_v.at[c]], bi_v.at[sl], bsem))

  def prep(i, carry):
    cc = i // (CH // L)
    off = (i % (CH // L)) * L
    uv = uidx_v[cc, pl.ds(off, L)]
    iv = iidx_v[cc, pl.ds(off, L)]
    sl = pl.ds(i * L, L)
    ih = jnp.where(iv >= jnp.int32(HALF), jnp.int32(1), jnp.int32(0))
    ilocal = iv - ih * jnp.int32(HALF)
    ur8_v[sl] = lax.bitwise_and(uv, -8)
    ir8_v[sl] = lax.bitwise_and(ilocal, -8)
    urr_v[sl] = lax.bitwise_and(uv, 7)
    irr_v[sl] = lax.bitwise_and(ilocal, 7)
    ihal_v[sl] = ih
    return carry
  lax.fori_loop(0, BPW // L, prep, 0)

  def fire(w, slot, usem, isem):
    u16 = ur8_v[pl.ds(w * WV, L)]
    i16 = ir8_v[pl.ds(w * WV, L)]
    ih16 = ihal_v[pl.ds(w * WV, L)]
    for j in range(L):
      ru = pl.multiple_of(u16[j], 8)
      ri = pl.multiple_of(i16[j], 8)
      ih = ih16[j]
      pltpu.async_copy(uf_hbm.at[0, pl.ds(ru, 8), pl.ds(0, D)],
                       pu_v.at[slot, j], usem)
      pltpu.async_copy(if_hbm.at[ih, pl.ds(ri, 8), pl.ds(0, D)],
                       qi_v.at[slot, j], isem)

  def drain(slot, usem, isem):
    for j in range(WV):
      pltpu.make_async_copy(uf_hbm.at[0, pl.ds(0, 8), pl.ds(0, D)],
                            pu_v.at[slot, j], usem).wait()
      pltpu.make_async_copy(if_hbm.at[0, pl.ds(0, 8), pl.ds(0, D)],
                            qi_v.at[slot, j], isem).wait()

  lane = lax.iota(jnp.int32, L)
  col_idx = lane * PAD

  def compute(w, slot):
    row0 = w * WV
    ur16 = urr_v[pl.ds(row0, L)]
    ir16 = irr_v[pl.ds(row0, L)]
    for j in range(L):
      ru = ur16[j]
      ri = ir16[j]
      acc = (pu_v[slot, j, ru, pl.ds(0, L)]
             * qi_v[slot, j, ri, pl.ds(0, L)])
      for k in range(1, D // L):
        acc = acc + (pu_v[slot, j, ru, pl.ds(k * L, L)]
                     * qi_v[slot, j, ri, pl.ds(k * L, L)])
      scr_v[pl.ds(j * PAD, L)] = acc
    dots0 = plsc.load_gather(scr_v, [col_idx])
    dots1 = plsc.load_gather(scr_v, [col_idx + 1])
    dots2 = plsc.load_gather(scr_v, [col_idx + 2])
    dots3 = plsc.load_gather(scr_v, [col_idx + 3])
    for l in range(4, L, 4):
      dots0 = dots0 + plsc.load_gather(scr_v, [col_idx + l])
      dots1 = dots1 + plsc.load_gather(scr_v, [col_idx + l + 1])
      dots2 = dots2 + plsc.load_gather(scr_v, [col_idx + l + 2])
      dots3 = dots3 + plsc.load_gather(scr_v, [col_idx + l + 3])
    dots = (dots0 + dots1) + (dots2 + dots3)
    sl = pl.ds(row0, L)
    rating = jnp.float32(_MU) + bu_v[sl] + bi_v[sl] + dots
    rating = jnp.minimum(jnp.maximum(rating, jnp.float32(1.0)),
                         jnp.float32(5.0))
    res_v[sl] = rating

  fire(0, 0, usem0, isem0)
  for h in bias_h:
    h.wait()

  def pair_body(p, carry):
    w0 = p * 2
    fire(w0 + 1, 1, usem1, isem1)
    drain(0, usem0, isem0)
    compute(w0, 0)

    @pl.when(p < NWV // 2 - 1)
    def _():
      fire(w0 + 2, 0, usem0, isem0)
    drain(1, usem1, isem1)
    compute(w0 + 1, 1)
    return carry

  lax.fori_loop(0, NWV // 2, pair_body, 0)

  pltpu.sync_copy(res_v, out_hbm.at[pl.ds(base, BPW)])


@jax.jit
def kernel(user_ids, item_ids, user_bias, item_bias, user_factors,
           item_factors):
  mesh = plsc.VectorSubcoreMesh(core_axis_name="c", subcore_axis_name="s")
  run = pl.kernel(
      _svd_body,
      out_type=jax.ShapeDtypeStruct((B,), jnp.float32),
      mesh=mesh,
      compiler_params=pltpu.CompilerParams(needs_layout_passes=False,
                                           use_tc_tiling_on_sc=True),
      scratch_types=[
          pltpu.VMEM((NCH, CH), jnp.int32),
          pltpu.VMEM((NCH, CH), jnp.int32),
          pltpu.VMEM((BPW,), jnp.int32),
          pltpu.VMEM((BPW,), jnp.int32),
          pltpu.VMEM((BPW,), jnp.int32),
          pltpu.VMEM((BPW,), jnp.int32),
          pltpu.VMEM((BPW,), jnp.int32),
          pltpu.VMEM((2, WV, 8, D), jnp.float32),
          pltpu.VMEM((2, WV, 8, D), jnp.float32),
          pltpu.VMEM((BPW,), jnp.float32),
          pltpu.VMEM((BPW,), jnp.float32),
          pltpu.VMEM((BPW,), jnp.float32),
          pltpu.VMEM((L * PAD,), jnp.float32),
          pltpu.SemaphoreType.DMA,
          pltpu.SemaphoreType.DMA,
          pltpu.SemaphoreType.DMA,
          pltpu.SemaphoreType.DMA,
          pltpu.SemaphoreType.DMA,
      ],
  )
  nu, ni = user_factors.shape[0], item_factors.shape[0]
  return run(user_ids.astype(jnp.int32), item_ids.astype(jnp.int32),
             user_bias.reshape(-1), item_bias.reshape(-1),
             user_factors.reshape(1, nu, D),
             item_factors.reshape(2, ni // 2, D))


# speedup vs baseline: 2.1795x; 1.0260x over previous
import jax
import jax.numpy as jnp
from jax import lax
from jax.experimental import pallas as pl
from jax.experimental.pallas import tpu as pltpu
from jax.experimental.pallas import tpu_sc as plsc

B = 16384
D = 64
NC, NS, L = 2, 16, 16
NW = NC * NS
BPW = B // NW
CH = 128
NCH = BPW // CH
WV = 16
NWV = BPW // WV
PAD = L + 1
HALF = 500000

_MU = 3.53


def _svd_body(uid_hbm, iid_hbm, ub_hbm, ib_hbm, uf_hbm, if_hbm, out_hbm,
              uidx_v, iidx_v, ur8_v, ir8_v, urr_v, irr_v, ihal_v,
              pu_v, qi_v, bu_v, bi_v, res_v, scr_v,
              bsem, usem0, isem0, usem1, isem1):
  wid = lax.axis_index("s") * NC + lax.axis_index("c")
  base = wid * BPW

  for c in range(NCH):
    pltpu.sync_copy(uid_hbm.at[pl.ds(base + c * CH, CH)], uidx_v.at[c])
    pltpu.sync_copy(iid_hbm.at[pl.ds(base + c * CH, CH)], iidx_v.at[c])

  bias_h = []
  for c in range(NCH):
    sl = pl.ds(c * CH, CH)
    bias_h.append(pltpu.async_copy(ub_hbm.at[uidx_v.at[c]], bu_v.at[sl], bsem))
    bias_h.append(pltpu.async_copy(ib_hbm.at[iidx_v.at[c]], bi_v.at[sl], bsem))

  def prep(i, carry):
    cc = i // (CH // L)
    off = (i % (CH // L)) * L
    uv = uidx_v[cc, pl.ds(off, L)]
    iv = iidx_v[cc, pl.ds(off, L)]
    sl = pl.ds(i * L, L)
    ih = jnp.where(iv >= jnp.int32(HALF), jnp.int32(1), jnp.int32(0))
    ilocal = iv - ih * jnp.int32(HALF)
    ur8_v[sl] = lax.bitwise_and(uv, -8)
    ir8_v[sl] = lax.bitwise_and(ilocal, -8)
    urr_v[sl] = lax.bitwise_and(uv, 7)
    irr_v[sl] = lax.bitwise_and(ilocal, 7)
    ihal_v[sl] = ih
    return carry
  lax.fori_loop(0, BPW // L, prep, 0)

  def fire(w, slot, usem, isem):
    u16 = ur8_v[pl.ds(w * WV, L)]
    i16 = ir8_v[pl.ds(w * WV, L)]
    ih16 = ihal_v[pl.ds(w * WV, L)]
    for j in range(L):
      ru = pl.multiple_of(u16[j], 8)
      ri = pl.multiple_of(i16[j], 8)
      ih = ih16[j]
      pltpu.async_copy(uf_hbm.at[pl.ds(ru, 8), pl.ds(0, D)],
                       pu_v.at[slot, j], usem)
      pltpu.async_copy(if_hbm.at[ih, pl.ds(ri, 8), pl.ds(0, D)],
                       qi_v.at[slot, j], isem)

  def drain(slot, usem, isem):
    for j in range(WV):
      pltpu.make_async_copy(uf_hbm.at[pl.ds(0, 8), pl.ds(0, D)],
                            pu_v.at[slot, j], usem).wait()
      pltpu.make_async_copy(if_hbm.at[0, pl.ds(0, 8), pl.ds(0, D)],
                            qi_v.at[slot, j], isem).wait()

  lane = lax.iota(jnp.int32, L)
  col_idx = lane * PAD

  def compute(w, slot):
    row0 = w * WV
    ur16 = urr_v[pl.ds(row0, L)]
    ir16 = irr_v[pl.ds(row0, L)]
    for j in range(L):
      ru = ur16[j]
      ri = ir16[j]
      acc = (pu_v[slot, j, ru, pl.ds(0, L)]
             * qi_v[slot, j, ri, pl.ds(0, L)])
      for k in range(1, D // L):
        acc = acc + (pu_v[slot, j, ru, pl.ds(k * L, L)]
                     * qi_v[slot, j, ri, pl.ds(k * L, L)])
      scr_v[pl.ds(j * PAD, L)] = acc
    dots0 = plsc.load_gather(scr_v, [col_idx])
    dots1 = plsc.load_gather(scr_v, [col_idx + 1])
    dots2 = plsc.load_gather(scr_v, [col_idx + 2])
    dots3 = plsc.load_gather(scr_v, [col_idx + 3])
    for l in range(4, L, 4):
      dots0 = dots0 + plsc.load_gather(scr_v, [col_idx + l])
      dots1 = dots1 + plsc.load_gather(scr_v, [col_idx + l + 1])
      dots2 = dots2 + plsc.load_gather(scr_v, [col_idx + l + 2])
      dots3 = dots3 + plsc.load_gather(scr_v, [col_idx + l + 3])
    dots = (dots0 + dots1) + (dots2 + dots3)
    sl = pl.ds(row0, L)
    rating = jnp.float32(_MU) + bu_v[sl] + bi_v[sl] + dots
    rating = jnp.minimum(jnp.maximum(rating, jnp.float32(1.0)),
                         jnp.float32(5.0))
    res_v[sl] = rating

  fire(0, 0, usem0, isem0)
  for h in bias_h:
    h.wait()

  def pair_body(p, carry):
    w0 = p * 2
    fire(w0 + 1, 1, usem1, isem1)
    drain(0, usem0, isem0)
    compute(w0, 0)

    @pl.when(p < NWV // 2 - 1)
    def _():
      fire(w0 + 2, 0, usem0, isem0)
    drain(1, usem1, isem1)
    compute(w0 + 1, 1)
    return carry

  lax.fori_loop(0, NWV // 2, pair_body, 0)

  pltpu.sync_copy(res_v, out_hbm.at[pl.ds(base, BPW)])


@jax.jit
def kernel(user_ids, item_ids, user_bias, item_bias, user_factors,
           item_factors):
  mesh = plsc.VectorSubcoreMesh(core_axis_name="c", subcore_axis_name="s")
  run = pl.kernel(
      _svd_body,
      out_type=jax.ShapeDtypeStruct((B,), jnp.float32),
      mesh=mesh,
      compiler_params=pltpu.CompilerParams(needs_layout_passes=False,
                                           use_tc_tiling_on_sc=True),
      scratch_types=[
          pltpu.VMEM((NCH, CH), jnp.int32),
          pltpu.VMEM((NCH, CH), jnp.int32),
          pltpu.VMEM((BPW,), jnp.int32),
          pltpu.VMEM((BPW,), jnp.int32),
          pltpu.VMEM((BPW,), jnp.int32),
          pltpu.VMEM((BPW,), jnp.int32),
          pltpu.VMEM((BPW,), jnp.int32),
          pltpu.VMEM((2, WV, 8, D), jnp.float32),
          pltpu.VMEM((2, WV, 8, D), jnp.float32),
          pltpu.VMEM((BPW,), jnp.float32),
          pltpu.VMEM((BPW,), jnp.float32),
          pltpu.VMEM((BPW,), jnp.float32),
          pltpu.VMEM((L * PAD,), jnp.float32),
          pltpu.SemaphoreType.DMA,
          pltpu.SemaphoreType.DMA,
          pltpu.SemaphoreType.DMA,
          pltpu.SemaphoreType.DMA,
          pltpu.SemaphoreType.DMA,
      ],
  )
  ni = item_factors.shape[0]
  return run(user_ids.astype(jnp.int32), item_ids.astype(jnp.int32),
             user_bias.reshape(-1), item_bias.reshape(-1),
             user_factors,
             item_factors.reshape(2, ni // 2, D))
